# baseline (speedup 1.0000x reference)
import jax
import jax.numpy as jnp
from jax.experimental import pallas as pl
from jax.experimental.pallas import tpu as pltpu

_STRIDE = 2
_OUT_G = 32
_NUM_SEG = _OUT_G ** 3
_C = 128

_P = 10000
_GROUPS = (3, 3, 3, 3, 2, 2)
_U = sum(_GROUPS)


def _scatter_kernel(seg_hbm, feats_ref, out_ref, seg_smem, sems):
    j = pl.program_id(0)
    n_tiles = pl.num_programs(0)
    slot = jax.lax.rem(j, 2)

    @pl.when(j == 0)
    def _():
        out_ref[...] = jnp.full(out_ref.shape, -jnp.inf, jnp.float32)
        pltpu.make_async_copy(seg_hbm.at[0], seg_smem.at[0], sems.at[0]).start()

    @pl.when(j + 1 < n_tiles)
    def _():
        nslot = jax.lax.rem(j + 1, 2)
        pltpu.make_async_copy(
            seg_hbm.at[j + 1], seg_smem.at[nslot], sems.at[nslot]
        ).start()

    pltpu.make_async_copy(seg_hbm.at[j], seg_smem.at[slot], sems.at[slot]).wait()

    def body(it, carry):
        base = it * _U
        off = 0
        for gs in _GROUPS:
            idxs = [base + off + i for i in range(gs)]
            off += gs
            ss = [seg_smem[slot, k] for k in idxs]
            vs = [
                jnp.maximum(out_ref[ss[i], 0], feats_ref[idxs[i], 0])
                for i in range(gs)
            ]
            for jj in range(1, gs):
                for ii in range(jj):
                    vs[jj] = jnp.maximum(
                        vs[jj], jnp.where(ss[ii] == ss[jj], vs[ii], -jnp.inf)
                    )
            for i in range(gs):
                out_ref[ss[i], 0] = vs[i]
        return carry

    jax.lax.fori_loop(0, _P // _U, body, 0)

    @pl.when(j == n_tiles - 1)
    def _():
        acc = out_ref[...]
        out_ref[...] = jnp.where(acc == -jnp.inf, jnp.float32(0.0), acc)


def kernel(feats, coords):
    n = feats.shape[0]
    cell = coords // _STRIDE
    seg = (cell[:, 0] * _OUT_G + cell[:, 1]) * _OUT_G + cell[:, 2]
    seg = seg.astype(jnp.int32)

    n_tiles = n // _P
    seg2d = seg.reshape(n_tiles, _P)
    feats3d = feats.reshape(n, 1, _C)

    pooled = pl.pallas_call(
        _scatter_kernel,
        out_shape=jax.ShapeDtypeStruct((_NUM_SEG, 1, _C), jnp.float32),
        grid=(n_tiles,),
        in_specs=[
            pl.BlockSpec(memory_space=pl.ANY),
            pl.BlockSpec((_P, 1, _C), lambda j: (j, 0, 0)),
        ],
        out_specs=pl.BlockSpec((_NUM_SEG, 1, _C), lambda j: (0, 0, 0)),
        scratch_shapes=[
            pltpu.SMEM((2, _P), jnp.int32),
            pltpu.SemaphoreType.DMA((2,)),
        ],
        compiler_params=pltpu.CompilerParams(
            dimension_semantics=("arbitrary",),
            vmem_limit_bytes=56 * 1024 * 1024,
        ),
        name="sparse_pool_scatter",
    )(seg2d, feats3d)
    return pooled.reshape(_NUM_SEG, _C)


# speedup vs baseline: 1.0507x; 1.0507x over previous
import jax
import jax.numpy as jnp
from jax.experimental import pallas as pl
from jax.experimental.pallas import tpu as pltpu

_STRIDE = 2
_OUT_G = 32
_NUM_SEG = _OUT_G ** 3
_C = 128

_P = 10000
_GROUPS = (2,) * 10
_U = sum(_GROUPS)


def _scatter_kernel(seg_hbm, feats_ref, out_ref, seg_smem, sems):
    j = pl.program_id(0)
    n_tiles = pl.num_programs(0)
    slot = jax.lax.rem(j, 2)

    @pl.when(j == 0)
    def _():
        out_ref[...] = jnp.full(out_ref.shape, -jnp.inf, jnp.float32)
        pltpu.make_async_copy(seg_hbm.at[0], seg_smem.at[0], sems.at[0]).start()

    @pl.when(j + 1 < n_tiles)
    def _():
        nslot = jax.lax.rem(j + 1, 2)
        pltpu.make_async_copy(
            seg_hbm.at[j + 1], seg_smem.at[nslot], sems.at[nslot]
        ).start()

    pltpu.make_async_copy(seg_hbm.at[j], seg_smem.at[slot], sems.at[slot]).wait()

    def body(it, carry):
        base = it * _U
        off = 0
        for gs in _GROUPS:
            idxs = [base + off + i for i in range(gs)]
            off += gs
            ss = [seg_smem[slot, k] for k in idxs]
            vs = [
                jnp.maximum(out_ref[ss[i], 0], feats_ref[idxs[i], 0])
                for i in range(gs)
            ]
            for jj in range(1, gs):
                for ii in range(jj):
                    vs[jj] = jnp.maximum(
                        vs[jj], jnp.where(ss[ii] == ss[jj], vs[ii], -jnp.inf)
                    )
            for i in range(gs):
                out_ref[ss[i], 0] = vs[i]
        return carry

    jax.lax.fori_loop(0, _P // _U, body, 0)

    @pl.when(j == n_tiles - 1)
    def _():
        acc = out_ref[...]
        out_ref[...] = jnp.where(acc == -jnp.inf, jnp.float32(0.0), acc)


def kernel(feats, coords):
    n = feats.shape[0]
    cell = coords // _STRIDE
    seg = (cell[:, 0] * _OUT_G + cell[:, 1]) * _OUT_G + cell[:, 2]
    seg = seg.astype(jnp.int32)

    n_tiles = n // _P
    seg2d = seg.reshape(n_tiles, _P)
    feats3d = feats.reshape(n, 1, _C)

    pooled = pl.pallas_call(
        _scatter_kernel,
        out_shape=jax.ShapeDtypeStruct((_NUM_SEG, 1, _C), jnp.float32),
        grid=(n_tiles,),
        in_specs=[
            pl.BlockSpec(memory_space=pl.ANY),
            pl.BlockSpec((_P, 1, _C), lambda j: (j, 0, 0)),
        ],
        out_specs=pl.BlockSpec((_NUM_SEG, 1, _C), lambda j: (0, 0, 0)),
        scratch_shapes=[
            pltpu.SMEM((2, _P), jnp.int32),
            pltpu.SemaphoreType.DMA((2,)),
        ],
        compiler_params=pltpu.CompilerParams(
            dimension_semantics=("arbitrary",),
            vmem_limit_bytes=56 * 1024 * 1024,
        ),
        name="sparse_pool_scatter",
    )(seg2d, feats3d)
    return pooled.reshape(_NUM_SEG, _C)
